# baseline (speedup 1.0000x reference)
import functools
import math

import jax
import jax.numpy as jnp
from jax import lax
from jax.experimental import pallas as pl
from jax.experimental.pallas import tpu as pltpu
from jax.experimental.pallas import tpu_sc as plsc


def _sc_gather(rows, din, nw, nch, ch):
    per_w = nch * ch
    mesh = plsc.VectorSubcoreMesh(core_axis_name="c", subcore_axis_name="s")

    @functools.partial(
        pl.kernel,
        mesh=mesh,
        out_type=jax.ShapeDtypeStruct((rows, din), jnp.float32),
        scratch_types=[
            pltpu.VMEM((nch, ch), jnp.int32),
            pltpu.VMEM((ch, din), jnp.float32),
            pltpu.VMEM((ch, din), jnp.float32),
            pltpu.SemaphoreType.DMA,
            pltpu.SemaphoreType.DMA,
        ],
    )
    def gather(idx_hbm, table_hbm, out_hbm, idx_v, buf0, buf1, sem0, sem1):
        info = plsc.get_sparse_core_info()
        wid = lax.axis_index("s") * info.num_cores + lax.axis_index("c")
        base = wid * per_w
        pltpu.sync_copy(idx_hbm.at[wid], idx_v)
        bufs = (buf0, buf1)
        sems = (sem0, sem1)
        cps = [None, None]
        cps[0] = pltpu.async_copy(table_hbm.at[idx_v.at[0]], buf0, sem0)
        for c in range(nch):
            nxt = c + 1
            if nxt < nch:
                cps[nxt % 2] = pltpu.async_copy(
                    table_hbm.at[idx_v.at[nxt]], bufs[nxt % 2], sems[nxt % 2])
            cps[c % 2].wait()
            pltpu.sync_copy(bufs[c % 2], out_hbm.at[pl.ds(base + c * ch, ch)])

    return gather


def _tc_project(rows_total, rows_chunk, row_blk, din, dout, chunk_idx, aliased):
    nblk = rows_chunk // row_blk
    base_blk = chunk_idx * nblk

    def body(*refs):
        x_ref, w_ref, o_ref = refs[0], refs[1], refs[-1]
        o_ref[...] = lax.dot_general(
            x_ref[...].astype(jnp.bfloat16), w_ref[...],
            (((1,), (1,)), ((), ())),
            preferred_element_type=jnp.float32)

    in_specs = [
        pl.BlockSpec((row_blk, din), lambda i: (i, 0)),
        pl.BlockSpec((dout, din), lambda i: (0, 0)),
    ]
    kwargs = {}
    if aliased:
        in_specs.append(pl.BlockSpec(memory_space=pl.ANY))
        kwargs["input_output_aliases"] = {2: 0}
    return pl.pallas_call(
        body,
        grid=(nblk,),
        in_specs=in_specs,
        out_specs=pl.BlockSpec((row_blk, dout), lambda i: (base_blk + i, 0)),
        out_shape=jax.ShapeDtypeStruct((rows_total, dout), jnp.float32),
        **kwargs)


def kernel(input, embed_weight, proj_weight):
    bsz, seq = input.shape
    _, din = embed_weight.shape
    dout = proj_weight.shape[0]
    scale = math.sqrt(float(dout))
    ntok = bsz * seq

    nchunks = 4
    nw = 32
    ch = 32
    row_blk = 512
    rows_c = ntok // nchunks
    per_w = rows_c // nw
    nch = per_w // ch

    idx = jnp.transpose(input).reshape(nchunks, nw, nch, ch)
    w_bf = (proj_weight * scale).astype(jnp.bfloat16)
    gather_fn = _sc_gather(rows_c, din, nw, nch, ch)

    out = None
    for k in range(nchunks):
        g = gather_fn(idx[k], embed_weight)
        mm = _tc_project(ntok, rows_c, row_blk, din, dout, k, aliased=k > 0)
        out = mm(g, w_bf) if k == 0 else mm(g, w_bf, out)
    return out.reshape(seq, bsz, dout)


# speedup vs baseline: 1.0322x; 1.0322x over previous
import functools
import math

import jax
import jax.numpy as jnp
from jax import lax
from jax.experimental import pallas as pl
from jax.experimental.pallas import tpu as pltpu
from jax.experimental.pallas import tpu_sc as plsc


def _sc_gather(rows, din, nw, nch, ch):
    per_w = nch * ch
    mesh = plsc.VectorSubcoreMesh(core_axis_name="c", subcore_axis_name="s")

    @functools.partial(
        pl.kernel,
        mesh=mesh,
        out_type=jax.ShapeDtypeStruct((rows, din), jnp.float32),
        scratch_types=[
            pltpu.VMEM((nch, ch), jnp.int32),
            pltpu.VMEM((ch, din), jnp.float32),
            pltpu.VMEM((ch, din), jnp.float32),
            pltpu.SemaphoreType.DMA,
            pltpu.SemaphoreType.DMA,
        ],
    )
    def gather(idx_hbm, table_hbm, out_hbm, idx_v, buf0, buf1, sem0, sem1):
        info = plsc.get_sparse_core_info()
        wid = lax.axis_index("s") * info.num_cores + lax.axis_index("c")
        base = wid * per_w
        pltpu.sync_copy(idx_hbm.at[wid], idx_v)
        bufs = (buf0, buf1)
        sems = (sem0, sem1)
        cps = [None, None]
        cps[0] = pltpu.async_copy(table_hbm.at[idx_v.at[0]], buf0, sem0)
        for c in range(nch):
            nxt = c + 1
            if nxt < nch:
                cps[nxt % 2] = pltpu.async_copy(
                    table_hbm.at[idx_v.at[nxt]], bufs[nxt % 2], sems[nxt % 2])
            cps[c % 2].wait()
            pltpu.sync_copy(bufs[c % 2], out_hbm.at[pl.ds(base + c * ch, ch)])

    return gather


def _tc_project(rows_total, rows_chunk, row_blk, din, dout, chunk_idx, aliased):
    nblk = rows_chunk // row_blk
    base_blk = chunk_idx * nblk

    def body(*refs):
        x_ref, w_ref, o_ref = refs[0], refs[1], refs[-1]
        o_ref[...] = lax.dot_general(
            x_ref[...].astype(jnp.bfloat16), w_ref[...],
            (((1,), (1,)), ((), ())),
            preferred_element_type=jnp.float32)

    in_specs = [
        pl.BlockSpec((row_blk, din), lambda i: (i, 0)),
        pl.BlockSpec((dout, din), lambda i: (0, 0)),
    ]
    kwargs = {}
    if aliased:
        in_specs.append(pl.BlockSpec(memory_space=pl.ANY))
        kwargs["input_output_aliases"] = {2: 0}
    return pl.pallas_call(
        body,
        grid=(nblk,),
        in_specs=in_specs,
        out_specs=pl.BlockSpec((row_blk, dout), lambda i: (base_blk + i, 0)),
        out_shape=jax.ShapeDtypeStruct((rows_total, dout), jnp.float32),
        **kwargs)


def kernel(input, embed_weight, proj_weight):
    bsz, seq = input.shape
    _, din = embed_weight.shape
    dout = proj_weight.shape[0]
    scale = math.sqrt(float(dout))
    ntok = bsz * seq

    nchunks = 1
    nw = 32
    ch = 32
    row_blk = 512
    rows_c = ntok // nchunks
    per_w = rows_c // nw
    nch = per_w // ch

    idx = jnp.transpose(input).reshape(nchunks, nw, nch, ch)
    w_bf = (proj_weight * scale).astype(jnp.bfloat16)
    gather_fn = _sc_gather(rows_c, din, nw, nch, ch)

    out = None
    for k in range(nchunks):
        g = gather_fn(idx[k], embed_weight)
        mm = _tc_project(ntok, rows_c, row_blk, din, dout, k, aliased=k > 0)
        out = mm(g, w_bf) if k == 0 else mm(g, w_bf, out)
    return out.reshape(seq, bsz, dout)


# speedup vs baseline: 1.8101x; 1.7537x over previous
import functools
import math

import jax
import jax.numpy as jnp
from jax import lax
from jax.experimental import pallas as pl
from jax.experimental.pallas import tpu as pltpu
from jax.experimental.pallas import tpu_sc as plsc


def _sc_gather(ntok, din, nw, nch, ch):
    per_w = nch * ch
    mesh = plsc.VectorSubcoreMesh(core_axis_name="c", subcore_axis_name="s")

    @functools.partial(
        pl.kernel,
        mesh=mesh,
        out_type=jax.ShapeDtypeStruct((ntok, din), jnp.float32),
        scratch_types=[
            pltpu.VMEM((nch, ch), jnp.int32),
            pltpu.VMEM((ch, din), jnp.float32),
            pltpu.VMEM((ch, din), jnp.float32),
            pltpu.SemaphoreType.DMA,
            pltpu.SemaphoreType.DMA,
        ],
    )
    def gather(idx_hbm, table_hbm, out_hbm, idx_v, buf0, buf1, sem0, sem1):
        info = plsc.get_sparse_core_info()
        wid = lax.axis_index("s") * info.num_cores + lax.axis_index("c")
        base = wid * per_w
        pltpu.sync_copy(idx_hbm.at[wid], idx_v)
        bufs = (buf0, buf1)
        sems = (sem0, sem1)
        cps = [None, None]
        cps[0] = pltpu.async_copy(table_hbm.at[idx_v.at[0]], buf0, sem0)
        for c in range(nch):
            nxt = c + 1
            if nxt < nch:
                cps[nxt % 2] = pltpu.async_copy(
                    table_hbm.at[idx_v.at[nxt]], bufs[nxt % 2], sems[nxt % 2])
            cps[c % 2].wait()
            pltpu.sync_copy(bufs[c % 2], out_hbm.at[pl.ds(base + c * ch, ch)])

    return gather


def _tc_project(bsz, seq, din, dout, rb, scale):
    sb = rb // bsz

    def body(x_ref, w_ref, o_ref):
        w = w_ref[...].astype(jnp.bfloat16)
        y = lax.dot_general(
            x_ref[...].astype(jnp.bfloat16), w,
            (((1,), (1,)), ((), ())),
            preferred_element_type=jnp.float32) * scale
        o_ref[...] = y.reshape(sb, bsz, dout)

    return pl.pallas_call(
        body,
        grid=(bsz * seq // rb,),
        in_specs=[
            pl.BlockSpec((rb, din), lambda i: (i, 0)),
            pl.BlockSpec((dout, din), lambda i: (0, 0)),
        ],
        out_specs=pl.BlockSpec((sb, bsz, dout), lambda i: (i, 0, 0)),
        out_shape=jax.ShapeDtypeStruct((seq, bsz, dout), jnp.float32),
    )


def kernel(input, embed_weight, proj_weight):
    bsz, seq = input.shape
    _, din = embed_weight.shape
    dout = proj_weight.shape[0]
    scale = math.sqrt(float(dout))
    ntok = bsz * seq

    nw = 32
    ch = 32
    per_w = ntok // nw
    nch = per_w // ch

    idx3 = jnp.transpose(input).reshape(nw, nch, ch)
    gathered = _sc_gather(ntok, din, nw, nch, ch)(idx3, embed_weight)
    return _tc_project(bsz, seq, din, dout, 512, scale)(gathered, proj_weight)


# speedup vs baseline: 1.9005x; 1.0499x over previous
import functools
import math

import jax
import jax.numpy as jnp
from jax import lax
from jax.experimental import pallas as pl
from jax.experimental.pallas import tpu as pltpu
from jax.experimental.pallas import tpu_sc as plsc


def _sc_gather(ntok, din, nw, nch, ch):
    per_w = nch * ch
    mesh = plsc.VectorSubcoreMesh(core_axis_name="c", subcore_axis_name="s")

    @functools.partial(
        pl.kernel,
        mesh=mesh,
        out_type=jax.ShapeDtypeStruct((ntok, din), jnp.float32),
        scratch_types=[
            pltpu.VMEM((nch, ch), jnp.int32),
            pltpu.VMEM((ch, din), jnp.float32),
            pltpu.VMEM((ch, din), jnp.float32),
            pltpu.SemaphoreType.DMA,
            pltpu.SemaphoreType.DMA,
        ],
    )
    def gather(idx_hbm, table_hbm, out_hbm, idx_v, buf0, buf1, sem0, sem1):
        info = plsc.get_sparse_core_info()
        wid = lax.axis_index("s") * info.num_cores + lax.axis_index("c")
        base = wid * per_w
        pltpu.sync_copy(idx_hbm.at[wid], idx_v)
        bufs = (buf0, buf1)
        sems = (sem0, sem1)
        cps = [None, None]
        cps[0] = pltpu.async_copy(table_hbm.at[idx_v.at[0]], buf0, sem0)
        for c in range(nch):
            nxt = c + 1
            if nxt < nch:
                cps[nxt % 2] = pltpu.async_copy(
                    table_hbm.at[idx_v.at[nxt]], bufs[nxt % 2], sems[nxt % 2])
            cps[c % 2].wait()
            pltpu.sync_copy(bufs[c % 2], out_hbm.at[pl.ds(base + c * ch, ch)])

    return gather


def _tc_project(bsz, seq, din, dout, rb, scale):
    sb = rb // bsz

    def body(x_ref, w_ref, o_ref):
        w = w_ref[...].astype(jnp.bfloat16)
        y = lax.dot_general(
            x_ref[...].astype(jnp.bfloat16), w,
            (((1,), (1,)), ((), ())),
            preferred_element_type=jnp.float32) * scale
        o_ref[...] = y.reshape(sb, bsz, dout)

    return pl.pallas_call(
        body,
        grid=(bsz * seq // rb,),
        in_specs=[
            pl.BlockSpec((rb, din), lambda i: (i, 0)),
            pl.BlockSpec((dout, din), lambda i: (0, 0)),
        ],
        out_specs=pl.BlockSpec((sb, bsz, dout), lambda i: (i, 0, 0)),
        out_shape=jax.ShapeDtypeStruct((seq, bsz, dout), jnp.float32),
    )


def kernel(input, embed_weight, proj_weight):
    bsz, seq = input.shape
    _, din = embed_weight.shape
    dout = proj_weight.shape[0]
    scale = math.sqrt(float(dout))
    ntok = bsz * seq

    nw = 32
    ch = 32
    per_w = ntok // nw
    nch = per_w // ch

    idx3 = jnp.transpose(input).reshape(nw, nch, ch)
    gathered = _sc_gather(ntok, din, nw, nch, ch)(idx3, embed_weight)
    return _tc_project(bsz, seq, din, dout, 1024, scale)(gathered, proj_weight)
